# baseline (speedup 1.0000x reference)
import dataclasses
import functools
import math

import jax
import jax.numpy as jnp
from jax import lax
from jax.experimental import pallas as pl
from jax.experimental.pallas import tpu as pltpu
from jax.experimental.pallas import tpu_sc as plsc

_LANES = 16
_NUM_WORKERS = 32
_IDX_PER_DMA = 128


def _table_body(n_pad, enc_ref, lt_ref, w_ref, b_ref, s_ref, e_ref, v_ref,
                out_ref, idx_ref):
    B, S, D = enc_ref.shape
    K = lt_ref.shape[0]
    nc = w_ref.shape[1]
    N = s_ref.shape[1]
    C = _LANES
    inv_row = B * K * S

    w = w_ref[...]
    w1 = w[0:D, :].astype(jnp.bfloat16)
    w2 = w[D:, :].astype(jnp.bfloat16)
    bias = b_ref[...]
    lenb = jnp.dot(lt_ref[...].astype(jnp.bfloat16), w2,
                   preferred_element_type=jnp.float32)

    def finish_rows(sc, rows):
        lane = lax.broadcasted_iota(jnp.int32, (rows, nc), 1)
        rowmax = jnp.max(sc, axis=1, keepdims=True)
        amax = jnp.min(jnp.where(sc == rowmax, lane, nc), axis=1,
                       keepdims=True).astype(jnp.float32)
        return jnp.concatenate(
            [sc, amax, jnp.zeros((rows, C - nc - 1), jnp.float32)], axis=1)

    blocks = []
    grp = 0
    for bi in range(B):
        e = enc_ref[bi]
        m = e
        for k in range(K):
            if k > 0:
                pad = jnp.broadcast_to(e[S - 1:S, :], (k, D))
                m = jnp.maximum(m, jnp.concatenate([e[k:, :], pad], axis=0))
            sc = (jnp.dot(m.astype(jnp.bfloat16), w1,
                          preferred_element_type=jnp.float32)
                  + lenb[k:k + 1, :]) + bias
            blocks.append(finish_rows(sc, S))
            if len(blocks) == 8:
                out_ref[pl.ds(grp * S, S), :] = jnp.concatenate(blocks, axis=1)
                blocks = []
                grp += 1

    inv = finish_rows(jnp.broadcast_to(bias, (1, nc)), 1)
    out_ref[pl.ds(grp * S, 1), :] = jnp.concatenate([inv] * 8, axis=1)

    s = s_ref[...]
    ev = e_ref[...]
    v = v_ref[...]
    kk = jnp.maximum(jnp.minimum(ev - s, K), 1) - 1
    st = jnp.maximum(jnp.minimum(s, S - 1), 0)
    bvec = lax.broadcasted_iota(jnp.int32, (B, N), 0)
    bk = bvec * K + kk
    idx = jnp.where(v, ((bk // 8) * S + st) * 8 + (bk % 8), inv_row)
    idx_ref[...] = jnp.full((B, n_pad), inv_row, jnp.int32)
    idx_ref[:, pl.ds(0, N)] = idx


def _make_gather_kernel(B, n_pad, nc):
    b_per_w = (B * n_pad) // _NUM_WORKERS
    n_dma = b_per_w // _IDX_PER_DMA
    w_per_b = _NUM_WORKERS // B
    mesh = plsc.VectorSubcoreMesh(core_axis_name="c", subcore_axis_name="s")
    cp = pltpu.CompilerParams(use_tc_tiling_on_sc=False)
    if "needs_layout_passes" in pltpu.CompilerParams.__dataclass_fields__:
        cp = dataclasses.replace(cp, needs_layout_passes=False)

    @functools.partial(
        pl.kernel, mesh=mesh,
        compiler_params=cp,
        out_type=(
            jax.ShapeDtypeStruct((B, n_pad * nc), jnp.float32),
            jax.ShapeDtypeStruct((B, n_pad), jnp.int32),
        ),
        scratch_types=[
            pltpu.VMEM((b_per_w,), jnp.int32),
            pltpu.VMEM((b_per_w, _LANES), jnp.float32),
            pltpu.VMEM((b_per_w * nc,), jnp.float32),
            pltpu.VMEM((b_per_w,), jnp.int32),
            pltpu.SemaphoreType.DMA,
        ],
    )
    def gather_kernel(table_hbm, i_hbm, scores_hbm, preds_hbm,
                      idx_v, rows_v, scores_v, preds_v, sem):
        wid = lax.axis_index("s") * 2 + lax.axis_index("c")
        bi = wid // w_per_b
        off = (wid % w_per_b) * b_per_w
        pltpu.sync_copy(i_hbm.at[bi, pl.ds(off, b_per_w)], idx_v)
        copies = [
            pltpu.async_copy(
                table_hbm.at[idx_v.at[pl.ds(j * _IDX_PER_DMA, _IDX_PER_DMA)]],
                rows_v.at[pl.ds(j * _IDX_PER_DMA, _IDX_PER_DMA)], sem)
            for j in range(n_dma)
        ]
        for c in copies:
            c.wait()

        lanes = lax.iota(jnp.int32, _LANES)
        colp = jnp.full((_LANES,), nc, jnp.int32)

        @pl.loop(0, b_per_w, step=_LANES)
        def _(i):
            n_vec = lanes + i
            preds_v[pl.ds(i, _LANES)] = plsc.load_gather(
                rows_v, [n_vec, colp]).astype(jnp.int32)
            dst = n_vec * nc
            for l in range(nc):
                vals = plsc.load_gather(rows_v,
                                        [n_vec, jnp.full((_LANES,), l,
                                                         jnp.int32)])
                plsc.store_scatter(scores_v, [dst + l], vals)

        pltpu.sync_copy(scores_v, scores_hbm.at[bi, pl.ds(off * nc,
                                                          b_per_w * nc)])
        pltpu.sync_copy(preds_v, preds_hbm.at[bi, pl.ds(off, b_per_w)])

    return gather_kernel


def kernel(encoded, n_words, span_starts, span_ends, span_valid,
           span_len_table, W, b):
    del n_words
    B, S, D = encoded.shape
    K = span_len_table.shape[0]
    nc = W.shape[1]
    N = span_starts.shape[1]
    C = _LANES

    align = (_NUM_WORKERS * _IDX_PER_DMA) // math.gcd(B, _NUM_WORKERS * _IDX_PER_DMA)
    n_pad = ((N + align - 1) // align) * align
    n_rows = B * K * S + 8

    table_p, idx = pl.pallas_call(
        functools.partial(_table_body, n_pad),
        out_shape=(
            jax.ShapeDtypeStruct((n_rows // 8, 8 * C), jnp.float32),
            jax.ShapeDtypeStruct((B, n_pad), jnp.int32),
        ),
    )(encoded, span_len_table, W, b.reshape(1, nc),
      span_starts, span_ends, span_valid)

    table = table_p.reshape(n_rows, C)

    scores_flat, preds_pad = _make_gather_kernel(B, n_pad, nc)(table, idx)

    scores = lax.slice(scores_flat.reshape(B, n_pad, nc),
                       (0, 0, 0), (B, N, nc))
    preds = lax.slice(preds_pad, (0, 0), (B, N))
    return scores, preds


# speedup vs baseline: 1.0884x; 1.0884x over previous
import dataclasses
import functools
import math

import jax
import jax.numpy as jnp
from jax import lax
from jax.experimental import pallas as pl
from jax.experimental.pallas import tpu as pltpu
from jax.experimental.pallas import tpu_sc as plsc

_LANES = 16
_NUM_WORKERS = 32
_IDX_PER_DMA = 128


def _table_body(n_pad, enc_ref, lt_ref, w_ref, b_ref, s_ref, e_ref, v_ref,
                out_ref, idx_ref):
    B, S, D = enc_ref.shape
    K = lt_ref.shape[0]
    nc = w_ref.shape[1]
    N = s_ref.shape[1]
    C = _LANES
    inv_row = B * K * S

    w = w_ref[...]
    w1 = w[0:D, :].astype(jnp.bfloat16)
    w2 = w[D:, :].astype(jnp.bfloat16)
    bias = b_ref[...]
    lenb = jnp.dot(lt_ref[...].astype(jnp.bfloat16), w2,
                   preferred_element_type=jnp.float32)

    def finish_rows(sc, rows):
        lane = lax.broadcasted_iota(jnp.int32, (rows, nc), 1)
        rowmax = jnp.max(sc, axis=1, keepdims=True)
        amax = jnp.min(jnp.where(sc == rowmax, lane, nc), axis=1,
                       keepdims=True).astype(jnp.float32)
        return jnp.concatenate(
            [sc, amax, jnp.zeros((rows, C - nc - 1), jnp.float32)], axis=1)

    blocks = []
    grp = 0
    for bi in range(B):
        e = enc_ref[bi]
        m = e
        for k in range(K):
            if k > 0:
                pad = jnp.broadcast_to(e[S - 1:S, :], (k, D))
                m = jnp.maximum(m, jnp.concatenate([e[k:, :], pad], axis=0))
            sc = (jnp.dot(m.astype(jnp.bfloat16), w1,
                          preferred_element_type=jnp.float32)
                  + lenb[k:k + 1, :]) + bias
            blocks.append(finish_rows(sc, S))
            if len(blocks) == 8:
                out_ref[pl.ds(grp * S, S), :] = jnp.concatenate(blocks, axis=1)
                blocks = []
                grp += 1

    inv = finish_rows(jnp.broadcast_to(bias, (1, nc)), 1)
    out_ref[pl.ds(grp * S, 1), :] = jnp.concatenate([inv] * 8, axis=1)

    s = s_ref[...]
    ev = e_ref[...]
    v = v_ref[...]
    kk = jnp.maximum(jnp.minimum(ev - s, K), 1) - 1
    st = jnp.maximum(jnp.minimum(s, S - 1), 0)
    bvec = lax.broadcasted_iota(jnp.int32, (B, N), 0)
    bk = bvec * K + kk
    idx = jnp.where(v, ((bk // 8) * S + st) * 8 + (bk % 8), inv_row)
    idx_ref[...] = jnp.full((B, n_pad), inv_row, jnp.int32)
    idx_ref[:, pl.ds(0, N)] = idx


def _make_gather_kernel(B, n_pad, nc):
    b_per_w = (B * n_pad) // _NUM_WORKERS
    n_dma = b_per_w // _IDX_PER_DMA
    w_per_b = _NUM_WORKERS // B
    mesh = plsc.VectorSubcoreMesh(core_axis_name="c", subcore_axis_name="s")
    cp = pltpu.CompilerParams(use_tc_tiling_on_sc=False)
    if "needs_layout_passes" in pltpu.CompilerParams.__dataclass_fields__:
        cp = dataclasses.replace(cp, needs_layout_passes=False)

    @functools.partial(
        pl.kernel, mesh=mesh,
        compiler_params=cp,
        out_type=(
            jax.ShapeDtypeStruct((B, n_pad, _LANES), jnp.float32),
            jax.ShapeDtypeStruct((B, n_pad), jnp.int32),
        ),
        scratch_types=[
            pltpu.VMEM((b_per_w,), jnp.int32),
            pltpu.VMEM((b_per_w, _LANES), jnp.float32),
            pltpu.VMEM((b_per_w,), jnp.int32),
            pltpu.SemaphoreType.DMA,
        ],
    )
    def gather_kernel(table_hbm, i_hbm, rows_hbm, preds_hbm,
                      idx_v, rows_v, preds_v, sem):
        wid = lax.axis_index("s") * 2 + lax.axis_index("c")
        bi = wid // w_per_b
        off = (wid % w_per_b) * b_per_w
        pltpu.sync_copy(i_hbm.at[bi, pl.ds(off, b_per_w)], idx_v)
        copies = [
            pltpu.async_copy(
                table_hbm.at[idx_v.at[pl.ds(j * _IDX_PER_DMA, _IDX_PER_DMA)]],
                rows_v.at[pl.ds(j * _IDX_PER_DMA, _IDX_PER_DMA)], sem)
            for j in range(n_dma)
        ]
        for c in copies:
            c.wait()

        lanes = lax.iota(jnp.int32, _LANES)
        col = jnp.full((_LANES,), nc, jnp.int32)

        @pl.loop(0, b_per_w, step=_LANES)
        def _(i):
            vals = plsc.load_gather(rows_v, [lanes + i, col])
            preds_v[pl.ds(i, _LANES)] = vals.astype(jnp.int32)

        pltpu.sync_copy(rows_v, rows_hbm.at[bi, pl.ds(off, b_per_w)])
        pltpu.sync_copy(preds_v, preds_hbm.at[bi, pl.ds(off, b_per_w)])

    return gather_kernel


def kernel(encoded, n_words, span_starts, span_ends, span_valid,
           span_len_table, W, b):
    del n_words
    B, S, D = encoded.shape
    K = span_len_table.shape[0]
    nc = W.shape[1]
    N = span_starts.shape[1]
    C = _LANES

    align = (_NUM_WORKERS * _IDX_PER_DMA) // math.gcd(B, _NUM_WORKERS * _IDX_PER_DMA)
    n_pad = ((N + align - 1) // align) * align
    n_rows = B * K * S + 8

    table_p, idx = pl.pallas_call(
        functools.partial(_table_body, n_pad),
        out_shape=(
            jax.ShapeDtypeStruct((n_rows // 8, 8 * C), jnp.float32),
            jax.ShapeDtypeStruct((B, n_pad), jnp.int32),
        ),
    )(encoded, span_len_table, W, b.reshape(1, nc),
      span_starts, span_ends, span_valid)

    table = table_p.reshape(n_rows, C)

    rows, preds_pad = _make_gather_kernel(B, n_pad, nc)(table, idx)

    scores = lax.slice(rows, (0, 0, 0), (B, N, nc))
    preds = lax.slice(preds_pad, (0, 0), (B, N))
    return scores, preds
